# baseline (speedup 1.0000x reference)
import functools
import jax
import jax.numpy as jnp
from jax import lax
from jax.experimental import pallas as pl
from jax.experimental.pallas import tpu as pltpu
from jax.experimental.pallas import tpu_sc as plsc

B, Q, C = 16, 20000, 92
N = B * Q
NW = 32
CHUNK = N // NW
T = 400
NT = CHUNK // T
NG = T // 16

_mesh = plsc.VectorSubcoreMesh(core_axis_name="c", subcore_axis_name="s")


def _shuffle(v, perm):
    return lax.gather(
        v,
        perm[:, None],
        lax.GatherDimensionNumbers(
            offset_dims=(), collapsed_slice_dims=(0,), start_index_map=(0,)
        ),
        slice_sizes=(1,),
        mode=lax.GatherScatterMode.PROMISE_IN_BOUNDS,
    )


@functools.partial(
    pl.kernel,
    mesh=_mesh,
    out_type=[
        jax.ShapeDtypeStruct((N,), jnp.float32),
        jax.ShapeDtypeStruct((N,), jnp.int32),
        jax.ShapeDtypeStruct((N * 4,), jnp.float32),
    ],
    scratch_types=[
        pltpu.VMEM((T * C + 16,), jnp.float32),
        pltpu.VMEM((T * 4,), jnp.float32),
        pltpu.VMEM((16,), jnp.float32),
        pltpu.VMEM((T,), jnp.float32),
        pltpu.VMEM((T,), jnp.int32),
        pltpu.VMEM((T * 4,), jnp.float32),
    ],
)
def _sc_post(logits_hbm, boxes_hbm, scalepat_hbm, scores_hbm, labels_hbm,
             oboxes_hbm, lg, bx, scv, sco, lbo, obo):
    wid = lax.axis_index("s") * 2 + lax.axis_index("c")
    qbase = wid * CHUNK
    pltpu.sync_copy(scalepat_hbm.at[wid], scv)
    scalev = scv[...]

    iota = lax.iota(jnp.int32, 16)
    neg_inf = jnp.full((16,), -jnp.inf, jnp.float32)
    big_i = jnp.full((16,), 1000, jnp.int32)
    mask_lt12 = iota < 12
    mask_lt11 = iota < 11
    lane11 = jnp.full((16,), 11, jnp.int32)
    perms = [iota ^ 8, iota ^ 4, iota ^ 2, iota ^ 1]
    cls_idx = [iota + 16 * j for j in range(6)]
    pair_lo = (iota & 2) == 0
    box_a = jnp.where(pair_lo, 1.0, 0.5).astype(jnp.float32)
    box_b = jnp.where(pair_lo, -0.5, 1.0).astype(jnp.float32)
    box_perm = iota ^ 2

    def tile_body(t, _):
        q0 = qbase + t * T
        pltpu.sync_copy(logits_hbm.at[pl.ds(q0 * C, T * C)], lg.at[pl.ds(0, T * C)])
        pltpu.sync_copy(boxes_hbm.at[pl.ds(q0 * 4, T * 4)], bx)

        def group_body(g, _):
            sc_acc = jnp.zeros((16,), jnp.float32)
            lb_acc = jnp.zeros((16,), jnp.int32)
            for q in range(16):
                base = (g * 16 + q) * C
                v = [lg[pl.ds(base + 16 * j, 16)] for j in range(6)]
                v5a = jnp.where(mask_lt12, v[5], neg_inf)
                v5b = jnp.where(mask_lt11, v[5], neg_inf)
                t4 = jnp.maximum(jnp.maximum(v[0], v[1]),
                                 jnp.maximum(v[2], v[3]))
                t4 = jnp.maximum(t4, v[4])
                m91 = jnp.maximum(t4, v5b)
                for p in perms:
                    m91 = jnp.maximum(m91, _shuffle(m91, p))
                c91 = _shuffle(v[5], lane11)
                m_all = jnp.maximum(m91, c91)
                va = v[:5] + [v5a]
                e = [jnp.exp(x - m_all) for x in va]
                s = (e[0] + e[1]) + (e[2] + e[3]) + (e[4] + e[5])
                for p in perms:
                    s = s + _shuffle(s, p)
                score = jnp.exp(m91 - m_all) / s
                vb = v[:5] + [v5b]
                cand = big_i
                for j in range(6):
                    cand = jnp.minimum(
                        cand, jnp.where(vb[j] == m91, cls_idx[j], big_i)
                    )
                for p in perms:
                    cand = jnp.minimum(cand, _shuffle(cand, p))
                sc_acc = jnp.where(iota == q, score, sc_acc)
                lb_acc = jnp.where(iota == q, cand, lb_acc)
            sco[pl.ds(g * 16, 16)] = sc_acc
            lbo[pl.ds(g * 16, 16)] = lb_acc
            return 0

        lax.fori_loop(0, NG, group_body, 0)

        def box_body(k, _):
            base = k * 16
            v = bx[pl.ds(base, 16)]
            vp = _shuffle(v, box_perm)
            obo[pl.ds(base, 16)] = (box_a * v + box_b * vp) * scalev
            return 0

        lax.fori_loop(0, T * 4 // 16, box_body, 0)

        pltpu.sync_copy(sco, scores_hbm.at[pl.ds(q0, T)])
        pltpu.sync_copy(lbo, labels_hbm.at[pl.ds(q0, T)])
        pltpu.sync_copy(obo, oboxes_hbm.at[pl.ds(q0 * 4, T * 4)])
        return 0

    lax.fori_loop(0, NT, tile_body, 0)


@jax.jit
def _run(logits_flat, boxes_flat, scale_pat):
    return _sc_post(logits_flat, boxes_flat, scale_pat)


def kernel(pred_logits, pred_boxes, target_sizes):
    ts = target_sizes.astype(jnp.float32)
    img_h = ts[:, 0]
    img_w = ts[:, 1]
    quad = jnp.stack([img_w, img_h, img_w, img_h], axis=1)
    scale_pat = jnp.repeat(jnp.tile(quad, (1, 4)), 2, axis=0)
    scores, labels, oboxes = _run(
        pred_logits.reshape(N * C), pred_boxes.reshape(N * 4), scale_pat
    )
    return (
        scores.reshape(B, Q),
        labels.reshape(B, Q),
        oboxes.reshape(B, Q, 4),
    )


# speedup vs baseline: 1.2198x; 1.2198x over previous
import functools
import jax
import jax.numpy as jnp
from jax import lax
from jax.experimental import pallas as pl
from jax.experimental.pallas import tpu as pltpu
from jax.experimental.pallas import tpu_sc as plsc

B, Q, C = 16, 20000, 92
N = B * Q
NW = 32
CHUNK = N // NW
T = 400
NT = CHUNK // T
NG = T // 16

_mesh = plsc.VectorSubcoreMesh(core_axis_name="c", subcore_axis_name="s")


def _shuffle(v, perm):
    return lax.gather(
        v,
        perm[:, None],
        lax.GatherDimensionNumbers(
            offset_dims=(), collapsed_slice_dims=(0,), start_index_map=(0,)
        ),
        slice_sizes=(1,),
        mode=lax.GatherScatterMode.PROMISE_IN_BOUNDS,
    )


@functools.partial(
    pl.kernel,
    mesh=_mesh,
    out_type=[
        jax.ShapeDtypeStruct((N,), jnp.float32),
        jax.ShapeDtypeStruct((N,), jnp.int32),
        jax.ShapeDtypeStruct((N * 4,), jnp.float32),
    ],
    scratch_types=[
        pltpu.VMEM((T, C), jnp.float32),
        pltpu.VMEM((T * 4,), jnp.float32),
        pltpu.VMEM((16,), jnp.float32),
        pltpu.VMEM((T,), jnp.float32),
        pltpu.VMEM((T,), jnp.int32),
        pltpu.VMEM((T * 4,), jnp.float32),
    ],
)
def _sc_post(logits_hbm, boxes_hbm, scalepat_hbm, scores_hbm, labels_hbm,
             oboxes_hbm, lg, bx, scv, sco, lbo, obo):
    wid = lax.axis_index("s") * 2 + lax.axis_index("c")
    qbase = wid * CHUNK
    pltpu.sync_copy(scalepat_hbm.at[wid], scv)
    scalev = scv[...]

    iota = lax.iota(jnp.int32, 16)
    neg_inf = jnp.full((16,), -jnp.inf, jnp.float32)
    big_i = jnp.full((16,), 1000, jnp.int32)
    mask_ge4 = iota >= 4
    mask_mid = mask_ge4 & (iota < 15)
    lane15 = jnp.full((16,), 15, jnp.int32)
    perms = [iota ^ 8, iota ^ 4, iota ^ 2, iota ^ 1]
    cls_idx = [iota + 16 * j for j in range(5)] + [iota + 76]
    pair_lo = (iota & 2) == 0
    box_a = jnp.where(pair_lo, 1.0, 0.5).astype(jnp.float32)
    box_b = jnp.where(pair_lo, -0.5, 1.0).astype(jnp.float32)
    box_perm = iota ^ 2

    def tile_body(t, _):
        q0 = qbase + t * T
        pltpu.sync_copy(logits_hbm.at[pl.ds(q0, T)], lg)
        pltpu.sync_copy(boxes_hbm.at[pl.ds(q0 * 4, T * 4)], bx)

        def group_body(g, _):
            sc_acc = jnp.zeros((16,), jnp.float32)
            lb_acc = jnp.zeros((16,), jnp.int32)
            for q in range(16):
                row = g * 16 + q
                v = [lg[row, pl.ds(16 * j, 16)] for j in range(5)]
                v.append(lg[row, pl.ds(76, 16)])
                v5a = jnp.where(mask_ge4, v[5], neg_inf)
                v5b = jnp.where(mask_mid, v[5], neg_inf)
                t4 = jnp.maximum(jnp.maximum(v[0], v[1]),
                                 jnp.maximum(v[2], v[3]))
                t4 = jnp.maximum(t4, v[4])
                m91 = jnp.maximum(t4, v5b)
                for p in perms:
                    m91 = jnp.maximum(m91, _shuffle(m91, p))
                c91 = _shuffle(v[5], lane15)
                m_all = jnp.maximum(m91, c91)
                va = v[:5] + [v5a]
                e = [jnp.exp(x - m_all) for x in va]
                s = (e[0] + e[1]) + (e[2] + e[3]) + (e[4] + e[5])
                for p in perms:
                    s = s + _shuffle(s, p)
                score = jnp.exp(m91 - m_all) / s
                vb = v[:5] + [v5b]
                cand = big_i
                for j in range(6):
                    cand = jnp.minimum(
                        cand, jnp.where(vb[j] == m91, cls_idx[j], big_i)
                    )
                for p in perms:
                    cand = jnp.minimum(cand, _shuffle(cand, p))
                sc_acc = jnp.where(iota == q, score, sc_acc)
                lb_acc = jnp.where(iota == q, cand, lb_acc)
            sco[pl.ds(g * 16, 16)] = sc_acc
            lbo[pl.ds(g * 16, 16)] = lb_acc
            return 0

        lax.fori_loop(0, NG, group_body, 0)

        def box_body(k, _):
            base = k * 16
            v = bx[pl.ds(base, 16)]
            vp = _shuffle(v, box_perm)
            obo[pl.ds(base, 16)] = (box_a * v + box_b * vp) * scalev
            return 0

        lax.fori_loop(0, T * 4 // 16, box_body, 0)

        pltpu.sync_copy(sco, scores_hbm.at[pl.ds(q0, T)])
        pltpu.sync_copy(lbo, labels_hbm.at[pl.ds(q0, T)])
        pltpu.sync_copy(obo, oboxes_hbm.at[pl.ds(q0 * 4, T * 4)])
        return 0

    lax.fori_loop(0, NT, tile_body, 0)


@jax.jit
def _run(logits_flat, boxes_flat, scale_pat):
    return _sc_post(logits_flat, boxes_flat, scale_pat)


def kernel(pred_logits, pred_boxes, target_sizes):
    ts = target_sizes.astype(jnp.float32)
    img_h = ts[:, 0]
    img_w = ts[:, 1]
    quad = jnp.stack([img_w, img_h, img_w, img_h], axis=1)
    scale_pat = jnp.repeat(jnp.tile(quad, (1, 4)), 2, axis=0)
    scores, labels, oboxes = _run(
        pred_logits.reshape(N, C), pred_boxes.reshape(N * 4), scale_pat
    )
    return (
        scores.reshape(B, Q),
        labels.reshape(B, Q),
        oboxes.reshape(B, Q, 4),
    )


# speedup vs baseline: 5.3146x; 4.3570x over previous
import functools
import jax
import jax.numpy as jnp
from jax import lax
from jax.experimental import pallas as pl
from jax.experimental.pallas import tpu as pltpu
from jax.experimental.pallas import tpu_sc as plsc

B, Q, C = 16, 20000, 92
CCH = 23
NCH = C // CCH
QW = 128
QT = 32
NQT = Q // QW + 1
NSLAB = 2 * NQT
NW = 32

_mesh = plsc.VectorSubcoreMesh(core_axis_name="c", subcore_axis_name="s")


def _shuffle(v, perm):
    return lax.gather(
        v,
        perm[:, None],
        lax.GatherDimensionNumbers(
            offset_dims=(), collapsed_slice_dims=(0,), start_index_map=(0,)
        ),
        slice_sizes=(1,),
        mode=lax.GatherScatterMode.PROMISE_IN_BOUNDS,
    )


@functools.partial(
    pl.kernel,
    mesh=_mesh,
    out_type=[
        jax.ShapeDtypeStruct((B, Q), jnp.float32),
        jax.ShapeDtypeStruct((B, Q), jnp.int32),
        jax.ShapeDtypeStruct((B, 4, Q), jnp.float32),
    ],
    scratch_types=[
        pltpu.VMEM((CCH, 8, QW), jnp.float32),
        pltpu.VMEM((8, 4, QW), jnp.float32),
        pltpu.VMEM((4, 16), jnp.float32),
        pltpu.VMEM((8, QW), jnp.float32),
        pltpu.VMEM((8, QW), jnp.int32),
        pltpu.VMEM((8, QW), jnp.float32),
        pltpu.VMEM((8, QW), jnp.int32),
        pltpu.VMEM((8, QW), jnp.float32),
        pltpu.VMEM((8, QW), jnp.float32),
        pltpu.VMEM((CCH, 8, QT), jnp.float32),
        pltpu.VMEM((8, 4, QT), jnp.float32),
        pltpu.VMEM((8, QT), jnp.float32),
        pltpu.VMEM((8, QT), jnp.int32),
    ],
)
def _sc_post(logits_hbm, boxes_hbm, scale_hbm, scores_hbm, labels_hbm,
             oboxes_hbm, lg, bxs, scl, sco, lbo, m91s, lbls, maccs, saccs,
             lg_t, bxs_t, sco_t, lbo_t):
    wid = lax.axis_index("s") * 2 + lax.axis_index("c")
    pltpu.sync_copy(scale_hbm, scl)
    scl_rows = [scl[j] for j in range(4)]

    iota = lax.iota(jnp.int32, 16)
    neg_inf = jnp.full((16,), -jnp.inf, jnp.float32)
    one_i = jnp.full((16,), 1, jnp.int32)

    def make_slab(qw, lgr, bxr, scor, lbor):
        ngrp = qw // 16

        def load_chunk(b0, q0, ch):
            pltpu.sync_copy(
                logits_hbm.at[pl.ds(ch * CCH, CCH), pl.ds(b0, 8),
                              pl.ds(q0, qw)],
                lgr,
            )

        def chunk_groups(ch):
            def row(bb, _):
                def group(u, _):
                    sl = pl.ds(u * 16, 16)
                    m91 = m91s[bb, sl]
                    lbl = lbls[bb, sl]
                    macc = maccs[bb, sl]
                    sacc = saccs[bb, sl]
                    cnt = jnp.full((16,), ch * CCH, jnp.int32)
                    mc = neg_inf
                    vals = []
                    for c in range(CCH):
                        v = lgr[c, bb, sl]
                        vals.append(v)
                        mc = jnp.maximum(mc, v)
                    is_last = ch == NCH - 1
                    for c in range(CCH - 1 if is_last else CCH):
                        v = vals[c]
                        upd = v > m91
                        m91 = jnp.where(upd, v, m91)
                        lbl = jnp.where(upd, cnt, lbl)
                        cnt = cnt + one_i
                    sc = jnp.zeros((16,), jnp.float32)
                    for c in range(CCH):
                        sc = sc + jnp.exp(vals[c] - mc)
                    m_new = jnp.maximum(macc, mc)
                    sacc = (sacc * jnp.exp(macc - m_new)
                            + sc * jnp.exp(mc - m_new))
                    m91s[bb, sl] = m91
                    lbls[bb, sl] = lbl
                    maccs[bb, sl] = m_new
                    saccs[bb, sl] = sacc
                    return 0

                lax.fori_loop(0, ngrp, group, 0)
                return 0

            lax.fori_loop(0, 8, row, 0)

        def finalize(b0):
            def row(bb, _):
                b = b0 + bb
                svec = [_shuffle(scl_rows[j], jnp.full((16,), b, jnp.int32))
                        for j in range(4)]

                def group(u, _):
                    sl = pl.ds(u * 16, 16)
                    m91 = m91s[bb, sl]
                    m_all = maccs[bb, sl]
                    s = saccs[bb, sl]
                    scor[bb, sl] = jnp.exp(m91 - m_all) / s
                    lbor[bb, sl] = lbls[bb, sl]
                    xc = bxr[bb, 0, sl]
                    yc = bxr[bb, 1, sl]
                    w = bxr[bb, 2, sl]
                    h = bxr[bb, 3, sl]
                    obx0 = (xc - 0.5 * w) * svec[0]
                    oby0 = (yc - 0.5 * h) * svec[1]
                    obx1 = (xc + 0.5 * w) * svec[2]
                    oby1 = (yc + 0.5 * h) * svec[3]
                    bxr[bb, 0, sl] = obx0
                    bxr[bb, 1, sl] = oby0
                    bxr[bb, 2, sl] = obx1
                    bxr[bb, 3, sl] = oby1
                    return 0

                lax.fori_loop(0, ngrp, group, 0)
                return 0

            lax.fori_loop(0, 8, row, 0)

        def init_state():
            def row(bb, _):
                def group(u, _):
                    sl = pl.ds(u * 16, 16)
                    m91s[bb, sl] = neg_inf
                    lbls[bb, sl] = jnp.zeros((16,), jnp.int32)
                    maccs[bb, sl] = neg_inf
                    saccs[bb, sl] = jnp.zeros((16,), jnp.float32)
                    return 0

                lax.fori_loop(0, ngrp, group, 0)
                return 0

            lax.fori_loop(0, 8, row, 0)

        def process(b0, q0):
            init_state()
            for ch in range(NCH):
                load_chunk(b0, q0, ch)
                chunk_groups(ch)
            finalize(b0)

        return process

    proc_full = make_slab(QW, lg, bxs, sco, lbo)
    proc_tail = make_slab(QT, lg_t, bxs_t, sco_t, lbo_t)

    def slab_body(i, _):
        sid = wid + i * NW

        @pl.when(sid < 2 * (NQT - 1))
        def _():
            b0 = (sid % 2) * 8
            q0 = (sid // 2) * QW
            pltpu.sync_copy(
                boxes_hbm.at[pl.ds(b0, 8), :, pl.ds(q0, QW)], bxs)
            proc_full(b0, q0)
            pltpu.sync_copy(sco, scores_hbm.at[pl.ds(b0, 8), pl.ds(q0, QW)])
            pltpu.sync_copy(lbo, labels_hbm.at[pl.ds(b0, 8), pl.ds(q0, QW)])
            pltpu.sync_copy(
                bxs, oboxes_hbm.at[pl.ds(b0, 8), :, pl.ds(q0, QW)])

        @pl.when((sid >= 2 * (NQT - 1)) & (sid < NSLAB))
        def _():
            b0 = (sid % 2) * 8
            q0 = (NQT - 1) * QW
            pltpu.sync_copy(
                boxes_hbm.at[pl.ds(b0, 8), :, pl.ds(q0, QT)], bxs_t)
            proc_tail(b0, q0)
            pltpu.sync_copy(sco_t, scores_hbm.at[pl.ds(b0, 8), pl.ds(q0, QT)])
            pltpu.sync_copy(lbo_t, labels_hbm.at[pl.ds(b0, 8), pl.ds(q0, QT)])
            pltpu.sync_copy(
                bxs_t, oboxes_hbm.at[pl.ds(b0, 8), :, pl.ds(q0, QT)])

        return 0

    lax.fori_loop(0, (NSLAB + NW - 1) // NW, slab_body, 0)


@jax.jit
def _run(logits_t, boxes_t, scale_rows):
    return _sc_post(logits_t, boxes_t, scale_rows)


def kernel(pred_logits, pred_boxes, target_sizes):
    ts = target_sizes.astype(jnp.float32)
    img_h = ts[:, 0]
    img_w = ts[:, 1]
    scale_rows = jnp.stack([img_w, img_h, img_w, img_h], axis=0)
    logits_t = jnp.transpose(pred_logits, (2, 0, 1))
    boxes_t = jnp.transpose(pred_boxes, (0, 2, 1))
    scores, labels, ob = _run(logits_t, boxes_t, scale_rows)
    return scores, labels, jnp.transpose(ob, (0, 2, 1))


# speedup vs baseline: 6.4833x; 1.2199x over previous
import functools
import jax
import jax.numpy as jnp
from jax import lax
from jax.experimental import pallas as pl
from jax.experimental.pallas import tpu as pltpu
from jax.experimental.pallas import tpu_sc as plsc

B, Q, C = 16, 20000, 92
CCH = 23
NCH = C // CCH
QW = 128
QT = 32
NQT = Q // QW + 1
NSLAB = 2 * NQT
NW = 32

_mesh = plsc.VectorSubcoreMesh(core_axis_name="c", subcore_axis_name="s")


def _shuffle(v, perm):
    return lax.gather(
        v,
        perm[:, None],
        lax.GatherDimensionNumbers(
            offset_dims=(), collapsed_slice_dims=(0,), start_index_map=(0,)
        ),
        slice_sizes=(1,),
        mode=lax.GatherScatterMode.PROMISE_IN_BOUNDS,
    )


@functools.partial(
    pl.kernel,
    mesh=_mesh,
    out_type=[
        jax.ShapeDtypeStruct((B, Q), jnp.float32),
        jax.ShapeDtypeStruct((B, Q), jnp.int32),
        jax.ShapeDtypeStruct((B, 4, Q), jnp.float32),
    ],
    scratch_types=[
        pltpu.VMEM((CCH, 8, QW), jnp.float32),
        pltpu.VMEM((CCH, 8, QW), jnp.float32),
        pltpu.SemaphoreType.DMA,
        pltpu.SemaphoreType.DMA,
        pltpu.VMEM((8, 4, QW), jnp.float32),
        pltpu.VMEM((4, 16), jnp.float32),
        pltpu.VMEM((8, QW), jnp.float32),
        pltpu.VMEM((8, QW), jnp.int32),
        pltpu.VMEM((8, QW), jnp.float32),
        pltpu.VMEM((8, QW), jnp.int32),
        pltpu.VMEM((8, QW), jnp.float32),
        pltpu.VMEM((8, QW), jnp.float32),
        pltpu.VMEM((CCH, 8, QT), jnp.float32),
        pltpu.VMEM((8, 4, QT), jnp.float32),
        pltpu.VMEM((8, QT), jnp.float32),
        pltpu.VMEM((8, QT), jnp.int32),
    ],
)
def _sc_post(logits_hbm, boxes_hbm, scale_hbm, scores_hbm, labels_hbm,
             oboxes_hbm, lg, lg2, sem0, sem1, bxs, scl, sco, lbo,
             m91s, lbls, maccs, saccs, lg_t, bxs_t, sco_t, lbo_t):
    wid = lax.axis_index("s") * 2 + lax.axis_index("c")
    pltpu.sync_copy(scale_hbm, scl)
    scl_rows = [scl[j] for j in range(4)]

    iota = lax.iota(jnp.int32, 16)
    neg_inf = jnp.full((16,), -jnp.inf, jnp.float32)
    one_i = jnp.full((16,), 1, jnp.int32)

    def make_slab(qw, lgbufs, bxr, scor, lbor):
        ngrp = qw // 16

        def chunk_groups(ch, lgr):
            def row(bb, _):
                def group(u, _):
                    sl = pl.ds(u * 16, 16)
                    m91 = m91s[bb, sl]
                    lbl = lbls[bb, sl]
                    macc = maccs[bb, sl]
                    sacc = saccs[bb, sl]
                    cnt = jnp.full((16,), ch * CCH, jnp.int32)
                    mc = neg_inf
                    vals = []
                    for c in range(CCH):
                        v = lgr[c, bb, sl]
                        vals.append(v)
                        mc = jnp.maximum(mc, v)
                    is_last = ch == NCH - 1
                    for c in range(CCH - 1 if is_last else CCH):
                        v = vals[c]
                        upd = v > m91
                        m91 = jnp.where(upd, v, m91)
                        lbl = jnp.where(upd, cnt, lbl)
                        cnt = cnt + one_i
                    sc = jnp.zeros((16,), jnp.float32)
                    for c in range(CCH):
                        sc = sc + jnp.exp(vals[c] - mc)
                    m_new = jnp.maximum(macc, mc)
                    sacc = (sacc * jnp.exp(macc - m_new)
                            + sc * jnp.exp(mc - m_new))
                    m91s[bb, sl] = m91
                    lbls[bb, sl] = lbl
                    maccs[bb, sl] = m_new
                    saccs[bb, sl] = sacc
                    return 0

                lax.fori_loop(0, ngrp, group, 0)
                return 0

            lax.fori_loop(0, 8, row, 0)

        def finalize(b0):
            def row(bb, _):
                b = b0 + bb
                svec = [_shuffle(scl_rows[j], jnp.full((16,), b, jnp.int32))
                        for j in range(4)]

                def group(u, _):
                    sl = pl.ds(u * 16, 16)
                    m91 = m91s[bb, sl]
                    m_all = maccs[bb, sl]
                    s = saccs[bb, sl]
                    scor[bb, sl] = jnp.exp(m91 - m_all) / s
                    lbor[bb, sl] = lbls[bb, sl]
                    xc = bxr[bb, 0, sl]
                    yc = bxr[bb, 1, sl]
                    w = bxr[bb, 2, sl]
                    h = bxr[bb, 3, sl]
                    obx0 = (xc - 0.5 * w) * svec[0]
                    oby0 = (yc - 0.5 * h) * svec[1]
                    obx1 = (xc + 0.5 * w) * svec[2]
                    oby1 = (yc + 0.5 * h) * svec[3]
                    bxr[bb, 0, sl] = obx0
                    bxr[bb, 1, sl] = oby0
                    bxr[bb, 2, sl] = obx1
                    bxr[bb, 3, sl] = oby1
                    return 0

                lax.fori_loop(0, ngrp, group, 0)
                return 0

            lax.fori_loop(0, 8, row, 0)

        def init_state():
            def row(bb, _):
                def group(u, _):
                    sl = pl.ds(u * 16, 16)
                    m91s[bb, sl] = neg_inf
                    lbls[bb, sl] = jnp.zeros((16,), jnp.int32)
                    maccs[bb, sl] = neg_inf
                    saccs[bb, sl] = jnp.zeros((16,), jnp.float32)
                    return 0

                lax.fori_loop(0, ngrp, group, 0)
                return 0

            lax.fori_loop(0, 8, row, 0)

        def src(b0, q0, ch):
            return logits_hbm.at[pl.ds(ch * CCH, CCH), pl.ds(b0, 8),
                                 pl.ds(q0, qw)]

        def process(b0, q0):
            init_state()
            if len(lgbufs) == 2:
                bufs, sems = lgbufs
                cps = [pltpu.async_copy(src(b0, q0, 0), bufs[0], sems[0])]
                for ch in range(NCH):
                    cps[ch].wait()
                    if ch + 1 < NCH:
                        cps.append(
                            pltpu.async_copy(
                                src(b0, q0, ch + 1),
                                bufs[(ch + 1) % 2],
                                sems[(ch + 1) % 2],
                            )
                        )
                    chunk_groups(ch, bufs[ch % 2])
            else:
                (lgr,) = lgbufs
                for ch in range(NCH):
                    pltpu.sync_copy(src(b0, q0, ch), lgr)
                    chunk_groups(ch, lgr)
            finalize(b0)

        return process

    proc_full = make_slab(QW, ([lg, lg2], [sem0, sem1]), bxs, sco, lbo)
    proc_tail = make_slab(QT, (lg_t,), bxs_t, sco_t, lbo_t)

    def slab_body(i, _):
        sid = wid + i * NW

        @pl.when(sid < 2 * (NQT - 1))
        def _():
            b0 = (sid % 2) * 8
            q0 = (sid // 2) * QW
            pltpu.sync_copy(
                boxes_hbm.at[pl.ds(b0, 8), :, pl.ds(q0, QW)], bxs)
            proc_full(b0, q0)
            pltpu.sync_copy(sco, scores_hbm.at[pl.ds(b0, 8), pl.ds(q0, QW)])
            pltpu.sync_copy(lbo, labels_hbm.at[pl.ds(b0, 8), pl.ds(q0, QW)])
            pltpu.sync_copy(
                bxs, oboxes_hbm.at[pl.ds(b0, 8), :, pl.ds(q0, QW)])

        @pl.when((sid >= 2 * (NQT - 1)) & (sid < NSLAB))
        def _():
            b0 = (sid % 2) * 8
            q0 = (NQT - 1) * QW
            pltpu.sync_copy(
                boxes_hbm.at[pl.ds(b0, 8), :, pl.ds(q0, QT)], bxs_t)
            proc_tail(b0, q0)
            pltpu.sync_copy(sco_t, scores_hbm.at[pl.ds(b0, 8), pl.ds(q0, QT)])
            pltpu.sync_copy(lbo_t, labels_hbm.at[pl.ds(b0, 8), pl.ds(q0, QT)])
            pltpu.sync_copy(
                bxs_t, oboxes_hbm.at[pl.ds(b0, 8), :, pl.ds(q0, QT)])

        return 0

    lax.fori_loop(0, (NSLAB + NW - 1) // NW, slab_body, 0)


@jax.jit
def _run(logits_t, boxes_t, scale_rows):
    return _sc_post(logits_t, boxes_t, scale_rows)


def kernel(pred_logits, pred_boxes, target_sizes):
    ts = target_sizes.astype(jnp.float32)
    img_h = ts[:, 0]
    img_w = ts[:, 1]
    scale_rows = jnp.stack([img_w, img_h, img_w, img_h], axis=0)
    logits_t = jnp.transpose(pred_logits, (2, 0, 1))
    boxes_t = jnp.transpose(pred_boxes, (0, 2, 1))
    scores, labels, ob = _run(logits_t, boxes_t, scale_rows)
    return scores, labels, jnp.transpose(ob, (0, 2, 1))
